# baseline (speedup 1.0000x reference)
import functools

import jax
import jax.numpy as jnp
from jax import lax
from jax.experimental import pallas as pl
from jax.experimental.pallas import tpu as pltpu
from jax.experimental.pallas import tpu_sc as plsc

_B = 8
_T1 = 1024
_T2 = 16
_IN = 128
_M = 64
_DEPTH = 4
_ROWS = _B * (_T1 + 2)
_ROWS_PAD = 8320
_HALF = _ROWS_PAD // 2
_ZB = 4100
_N = _B * _T1
_NS = 16
_NPC = _N // 2
_NODES_PER_S = _NPC // _NS
_CHUNK_NODES = 8
_CHUNK_ROWS = _CHUNK_NODES * _T2
_NCHUNK = _NODES_PER_S // _CHUNK_NODES
_SUBS = 4
_TWW = 128
_HI = jnp.int32(-65536)


def _sigmoid(x):
    return jax.nn.sigmoid(x)


def _renorm_scale(x):
    n = jnp.sqrt(jnp.sum(x * x, axis=-1, keepdims=True))
    return jnp.where(n > 2.0, 2.0 / (n + 1e-7), 1.0)


def _stage_a_body(te_ref, trees_ref, cm_ref, wx_ref, bx_ref, bhiou_ref,
                  whf_ref, bhf_ref,
                  xiou_ref, xfsub_ref, midxa_ref, midxb_ref,
                  th_ref, yh_ref, tc_ref):
    m = _M
    te = te_ref[0]
    x = lax.dot_general(te, wx_ref[...], (((1,), (1,)), ((), ())),
                        preferred_element_type=jnp.float32) + bx_ref[0]
    xiou_ref[0] = x[:, :3 * m]
    xfsub_ref[0] = x[:_T2, 3 * m:]
    bh = bhiou_ref[0]
    i = _sigmoid(x[:, :m] + bh[:m])
    o = _sigmoid(x[:, m:2 * m] + bh[m:2 * m])
    u = jnp.tanh(x[:, 2 * m:3 * m] + bh[2 * m:3 * m])
    c = i * u
    h = o * jnp.tanh(c)
    th = h * _renorm_scale(h)
    tc = c * _renorm_scale(c)
    yh = lax.dot_general(th, whf_ref[...], (((1,), (1,)), ((), ())),
                         preferred_element_type=jnp.float32) + bhf_ref[0]
    th_ref[0] = th
    yh_ref[0] = yh
    tc_ref[0] = tc
    t = jnp.where(cm_ref[0] > 0.0, trees_ref[0], 0)
    midxa_ref[0] = jnp.where(t < _HALF, t, 0)
    midxb_ref[0] = jnp.where(t >= _HALF, t - _HALF, _ZB)


def _stage_bc_body(make_table, hsa_ref, fca_ref, hsb_ref, fcb_ref, xiou_ref,
                   whiou_ref, bhiou_ref, whf_ref, bhf_ref, *out_refs):
    m = _M
    hs = hsa_ref[0] + hsb_ref[0]
    fc = fca_ref[0] + fcb_ref[0]
    s = xiou_ref[0] + lax.dot_general(
        hs, whiou_ref[...], (((1,), (1,)), ((), ())),
        preferred_element_type=jnp.float32) + bhiou_ref[0]
    i = _sigmoid(s[:, :m])
    o = _sigmoid(s[:, m:2 * m])
    u = jnp.tanh(s[:, 2 * m:])
    c = i * u + fc
    h = o * jnp.tanh(c)
    if make_table:
        th_ref, yh_ref, tc_ref = out_refs
        th = h * _renorm_scale(h)
        tc = c * _renorm_scale(c)
        yh = lax.dot_general(th, whf_ref[...], (((1,), (1,)), ((), ())),
                             preferred_element_type=jnp.float32) + bhf_ref[0]
        th_ref[0] = th
        yh_ref[0] = yh
        tc_ref[0] = tc
    else:
        out_refs[0][0] = h


def _full(shape):
    return pl.BlockSpec(shape, lambda b: (0,) * len(shape))


def _batched(shape):
    return pl.BlockSpec((1,) + shape, lambda b: (b,) + (0,) * len(shape))


_stage_a = pl.pallas_call(
    _stage_a_body,
    grid=(_B,),
    in_specs=[
        _batched((_T1, _IN)),
        _batched((1, _T1 * _T2)),
        _batched((1, _T1 * _T2)),
        _full((4 * _M, _IN)),
        _full((1, 4 * _M)),
        _full((1, 3 * _M)),
        _full((_M, _M)),
        _full((1, _M)),
    ],
    out_specs=[
        _batched((_T1, 3 * _M)),
        _batched((_T2, _M)),
        _batched((1, _T1 * _T2)),
        _batched((1, _T1 * _T2)),
        _batched((_T1, _M)),
        _batched((_T1, _M)),
        _batched((_T1, _M)),
    ],
    out_shape=[
        jax.ShapeDtypeStruct((_B, _T1, 3 * _M), jnp.float32),
        jax.ShapeDtypeStruct((_B, _T2, _M), jnp.float32),
        jax.ShapeDtypeStruct((_B, 1, _T1 * _T2), jnp.int32),
        jax.ShapeDtypeStruct((_B, 1, _T1 * _T2), jnp.int32),
        jax.ShapeDtypeStruct((_B, _T1, _M), jnp.float32),
        jax.ShapeDtypeStruct((_B, _T1, _M), jnp.float32),
        jax.ShapeDtypeStruct((_B, _T1, _M), jnp.float32),
    ],
)

_bc_in_specs = [
    _batched((_T1, _M)),
    _batched((_T1, _M)),
    _batched((_T1, _M)),
    _batched((_T1, _M)),
    _batched((_T1, 3 * _M)),
    _full((3 * _M, _M)),
    _full((1, 3 * _M)),
    _full((_M, _M)),
    _full((1, _M)),
]

_stage_b = pl.pallas_call(
    functools.partial(_stage_bc_body, True),
    grid=(_B,),
    in_specs=_bc_in_specs,
    out_specs=[
        _batched((_T1, _M)),
        _batched((_T1, _M)),
        _batched((_T1, _M)),
    ],
    out_shape=[
        jax.ShapeDtypeStruct((_B, _T1, _M), jnp.float32),
        jax.ShapeDtypeStruct((_B, _T1, _M), jnp.float32),
        jax.ShapeDtypeStruct((_B, _T1, _M), jnp.float32),
    ],
)

_stage_c = pl.pallas_call(
    functools.partial(_stage_bc_body, False),
    grid=(_B,),
    in_specs=_bc_in_specs,
    out_specs=[_batched((_T1, _M))],
    out_shape=[jax.ShapeDtypeStruct((_B, _T1, _M), jnp.float32)],
)


def _unpack2(wv):
    lo = lax.bitcast_convert_type(lax.shift_left(wv, 16), jnp.float32)
    hi = lax.bitcast_convert_type(jnp.bitwise_and(wv, _HI), jnp.float32)
    return lo, hi


def _pack2(a, b):
    ua = lax.bitcast_convert_type(a, jnp.int32)
    ta = ua + 0x7FFF + jnp.bitwise_and(lax.shift_right_logical(ua, 16), 1)
    ub = lax.bitcast_convert_type(b, jnp.int32)
    tb = ub + 0x7FFF + jnp.bitwise_and(lax.shift_right_logical(ub, 16), 1)
    return jnp.bitwise_or(lax.shift_right_logical(ta, 16),
                          jnp.bitwise_and(tb, _HI))


def _sc_gather_body(table_hbm, midx_hbm, xf_hbm, out_hbm,
                    idx_v, xf_v, rows_v, out_v, spm, sem):
    s_id = lax.axis_index("s")
    c_id = lax.axis_index("c")
    b = s_id // 4
    pltpu.sync_copy(table_hbm.at[c_id, pl.ds(s_id * 256, 256)],
                    spm.at[pl.ds(s_id * 256, 256)])

    @pl.when(s_id == 0)
    def _():
        pltpu.sync_copy(table_hbm.at[c_id, pl.ds(4096, _HALF - 4096)],
                        spm.at[pl.ds(4096, _HALF - 4096)])

    pltpu.sync_copy(midx_hbm.at[c_id, s_id], idx_v)
    pltpu.sync_copy(xf_hbm.at[b], xf_v)
    plsc.subcore_barrier()

    def compute_chunk(g, buf):
        def node_body(n8, _):
            def child_body(k, acc):
                row = n8 * _T2 + k
                new = list(acc)
                for blk in range(2):
                    lo, hi = _unpack2(buf[row, pl.ds(16 * blk, 16)])
                    new[2 * blk] = acc[2 * blk] + lo
                    new[2 * blk + 1] = acc[2 * blk + 1] + hi
                ys, cs = [], []
                for blk in range(2, 4):
                    ys.extend(_unpack2(buf[row, pl.ds(16 * blk, 16)]))
                for blk in range(4, 6):
                    cs.extend(_unpack2(buf[row, pl.ds(16 * blk, 16)]))
                for s in range(4):
                    xv = xf_v[k, pl.ds(16 * s, 16)]
                    f = 1.0 / (1.0 + jnp.exp(-(ys[s] + xv)))
                    new[4 + s] = acc[4 + s] + f * cs[s]
                return tuple(new)

            zero = jnp.zeros((16,), jnp.float32)
            acc = lax.fori_loop(0, _T2, child_body, (zero,) * 8)
            node = g * _CHUNK_NODES + n8
            for s in range(4):
                out_v[node, pl.ds(16 * s, 16)] = _pack2(acc[s], acc[4 + s])
            return 0

        lax.fori_loop(0, _CHUNK_NODES, node_body, 0)

    sub = _CHUNK_ROWS // _SUBS
    copies = [[None] * _SUBS, [None] * _SUBS]

    def fire(g, buf):
        for s in range(_SUBS):
            copies[buf][s] = pltpu.async_copy(
                spm.at[idx_v.at[g, pl.ds(s * sub, sub)]],
                rows_v.at[buf, pl.ds(s * sub, sub)], sem.at[buf])

    fire(0, 0)
    for g in range(_NCHUNK):
        cur = g % 2
        if g + 1 < _NCHUNK:
            fire(g + 1, (g + 1) % 2)
        for s in range(_SUBS):
            copies[cur][s].wait()
        compute_chunk(g, rows_v.at[cur])
    pltpu.sync_copy(out_v,
                    out_hbm.at[c_id, pl.ds(s_id * _NODES_PER_S, _NODES_PER_S)])


@functools.cache
def _get_sc_gather():
    return functools.partial(
        pl.kernel,
        mesh=plsc.VectorSubcoreMesh(core_axis_name="c", subcore_axis_name="s"),
        out_type=jax.ShapeDtypeStruct((2, _NPC, _M), jnp.int32),
        scratch_types=[
            pltpu.VMEM((_NCHUNK, _CHUNK_ROWS), jnp.int32),
            pltpu.VMEM((_T2, _M), jnp.float32),
            pltpu.VMEM((2, _CHUNK_ROWS, _TWW), jnp.int32),
            pltpu.VMEM((_NODES_PER_S, _M), jnp.int32),
            pltpu.VMEM_SHARED((_HALF, _TWW), jnp.int32),
            pltpu.SemaphoreType.DMA((2,)),
        ],
    )(_sc_gather_body)


def _build_table(th, yh, tc):
    zpad = jnp.zeros((_B, _T1, _M), jnp.float32)
    row = jnp.concatenate([th, yh, tc, zpad], axis=-1)
    pad = jnp.zeros((_B, 2, 4 * _M), jnp.float32)
    tab = jnp.concatenate([pad, row], axis=1).reshape(_ROWS, 4 * _M)
    ztail = jnp.zeros((_ROWS_PAD - _ROWS, 4 * _M), jnp.float32)
    tab = jnp.concatenate([tab, ztail], axis=0).astype(jnp.bfloat16)
    pairs = tab.reshape(_ROWS_PAD, 8, 2, 16).transpose(0, 1, 3, 2)
    packed = lax.bitcast_convert_type(pairs, jnp.int32).reshape(_ROWS_PAD,
                                                                _TWW)
    return packed.reshape(2, _HALF, _TWW)


def _unpack_out(lo_i32, hi_i32):
    out_i32 = jnp.concatenate([lo_i32, hi_i32], axis=1)
    pr = lax.bitcast_convert_type(out_i32, jnp.bfloat16)
    hs = pr[..., 0].astype(jnp.float32).reshape(2, _B, _T1, _M)
    fc = pr[..., 1].astype(jnp.float32).reshape(2, _B, _T1, _M)
    return hs[0], fc[0], hs[1], fc[1]


def kernel(token_encodings, trees, child_mask, max_depth,
           Wx, bx, Wh_iou, bh_iou, Wh_f, bh_f):
    del max_depth
    trees_f = trees.reshape(_B, 1, _T1 * _T2).astype(jnp.int32)
    cm_f = child_mask.reshape(_B, 1, _T1 * _T2)
    bx2 = bx.reshape(1, 4 * _M)
    bhiou2 = bh_iou.reshape(1, 3 * _M)
    bhf2 = bh_f.reshape(1, _M)

    x_iou, xf_sub, midxa, midxb, th, yh, tc = _stage_a(
        token_encodings, trees_f, cm_f, Wx, bx2, bhiou2, Wh_f, bhf2)
    table = _build_table(th, yh, tc)
    midx_h = [jnp.stack([m.reshape(2, _NS, _NCHUNK, _CHUNK_ROWS)[h]
                         for m in (midxa, midxb)])
              for h in range(2)]
    xf_h = [xf_sub[h * 4:(h + 1) * 4] for h in range(2)]

    sc_gather = _get_sc_gather()
    for level in range(1, _DEPTH):
        parts = [sc_gather(table, midx_h[h], xf_h[h]) for h in range(2)]
        hsa, fca, hsb, fcb = _unpack_out(*parts)
        if level < _DEPTH - 1:
            th, yh, tc = _stage_b(hsa, fca, hsb, fcb, x_iou,
                                  Wh_iou, bhiou2, Wh_f, bhf2)
            table = _build_table(th, yh, tc)
        else:
            (h,) = _stage_c(hsa, fca, hsb, fcb, x_iou,
                            Wh_iou, bhiou2, Wh_f, bhf2)
    return h


# speedup vs baseline: 1.2692x; 1.2692x over previous
import functools

import jax
import jax.numpy as jnp
from jax import lax
from jax.experimental import pallas as pl
from jax.experimental.pallas import tpu as pltpu
from jax.experimental.pallas import tpu_sc as plsc

_B = 8
_T1 = 1024
_T2 = 16
_IN = 128
_M = 64
_DEPTH = 4
_ROWS = _B * (_T1 + 2)
_ROWS_PAD = 8320
_HALF = _ROWS_PAD // 2
_ZB = 4100
_N = _B * _T1
_NS = 16
_NPC = _N // 2
_NODES_PER_S = _NPC // _NS
_CHUNK_NODES = 8
_CHUNK_ROWS = _CHUNK_NODES * _T2
_NCHUNK = _NODES_PER_S // _CHUNK_NODES
_SUBS = 4
_TWW = 128
_HI = -65536


def _sigmoid(x):
    return jax.nn.sigmoid(x)


def _renorm_scale(x):
    n = jnp.sqrt(jnp.sum(x * x, axis=-1, keepdims=True))
    return jnp.where(n > 2.0, 2.0 / (n + 1e-7), 1.0)


def _tc_pack_words(y):
    def rne(u):
        return u + 0x7FFF + jnp.bitwise_and(lax.shift_right_logical(u, 16), 1)
    words = []
    for k in range(8):
        ua = rne(lax.bitcast_convert_type(y[:, 32 * k:32 * k + 16],
                                          jnp.int32))
        ub = rne(lax.bitcast_convert_type(y[:, 32 * k + 16:32 * k + 32],
                                          jnp.int32))
        words.append(jnp.bitwise_or(lax.shift_right_logical(ua, 16),
                                    jnp.bitwise_and(ub, _HI)))
    return jnp.concatenate(words, axis=-1)


def _tc_unpack_words(wv):
    lo = lax.bitcast_convert_type(lax.shift_left(wv, 16), jnp.float32)
    hi = lax.bitcast_convert_type(jnp.bitwise_and(wv, _HI), jnp.float32)
    return lo, hi


def _stage_a_body(te_ref, trees_ref, cm_ref, wx_ref, bx_ref, bhiou_ref,
                  whf_ref, bhf_ref,
                  xiou_ref, xfsub_ref, midxa_ref, midxb_ref, tab_ref):
    m = _M
    te = te_ref[0]
    x = lax.dot_general(te, wx_ref[...], (((1,), (1,)), ((), ())),
                        preferred_element_type=jnp.float32) + bx_ref[0]
    xiou_ref[0] = x[:, :3 * m]
    xfsub_ref[0] = x[:_T2, 3 * m:]
    bh = bhiou_ref[0]
    i = _sigmoid(x[:, :m] + bh[:m])
    o = _sigmoid(x[:, m:2 * m] + bh[m:2 * m])
    u = jnp.tanh(x[:, 2 * m:3 * m] + bh[2 * m:3 * m])
    c = i * u
    h = o * jnp.tanh(c)
    th = h * _renorm_scale(h)
    tc = c * _renorm_scale(c)
    yh = lax.dot_general(th, whf_ref[...], (((1,), (1,)), ((), ())),
                         preferred_element_type=jnp.float32) + bhf_ref[0]
    zp = jnp.zeros((_T1, _M), jnp.float32)
    tab_ref[0] = _tc_pack_words(jnp.concatenate([th, yh, tc, zp], axis=-1))
    t = jnp.where(cm_ref[0] > 0.0, trees_ref[0], 0)
    midxa_ref[0] = jnp.where(t < _HALF, t, 0)
    midxb_ref[0] = jnp.where(t >= _HALF, t - _HALF, _ZB)


def _stage_bc_body(make_table, p0_ref, p1_ref, xiou_ref,
                   whiou_ref, bhiou_ref, whf_ref, bhf_ref, *out_refs):
    m = _M
    hs0, fc0 = _tc_unpack_words(p0_ref[0])
    hs1, fc1 = _tc_unpack_words(p1_ref[0])
    hs = hs0 + hs1
    fc = fc0 + fc1
    s = xiou_ref[0] + lax.dot_general(
        hs, whiou_ref[...], (((1,), (1,)), ((), ())),
        preferred_element_type=jnp.float32) + bhiou_ref[0]
    i = _sigmoid(s[:, :m])
    o = _sigmoid(s[:, m:2 * m])
    u = jnp.tanh(s[:, 2 * m:])
    c = i * u + fc
    h = o * jnp.tanh(c)
    if make_table:
        th = h * _renorm_scale(h)
        tc = c * _renorm_scale(c)
        yh = lax.dot_general(th, whf_ref[...], (((1,), (1,)), ((), ())),
                             preferred_element_type=jnp.float32) + bhf_ref[0]
        zp = jnp.zeros((_T1, _M), jnp.float32)
        out_refs[0][0] = _tc_pack_words(
            jnp.concatenate([th, yh, tc, zp], axis=-1))
    else:
        out_refs[0][0] = h


def _full(shape):
    return pl.BlockSpec(shape, lambda b: (0,) * len(shape))


def _batched(shape):
    return pl.BlockSpec((1,) + shape, lambda b: (b,) + (0,) * len(shape))


_stage_a = pl.pallas_call(
    _stage_a_body,
    grid=(_B,),
    in_specs=[
        _batched((_T1, _IN)),
        _batched((1, _T1 * _T2)),
        _batched((1, _T1 * _T2)),
        _full((4 * _M, _IN)),
        _full((1, 4 * _M)),
        _full((1, 3 * _M)),
        _full((_M, _M)),
        _full((1, _M)),
    ],
    out_specs=[
        _batched((_T1, 3 * _M)),
        _batched((_T2, _M)),
        _batched((1, _T1 * _T2)),
        _batched((1, _T1 * _T2)),
        _batched((_T1, 2 * _M)),
    ],
    out_shape=[
        jax.ShapeDtypeStruct((_B, _T1, 3 * _M), jnp.float32),
        jax.ShapeDtypeStruct((_B, _T2, _M), jnp.float32),
        jax.ShapeDtypeStruct((_B, 1, _T1 * _T2), jnp.int32),
        jax.ShapeDtypeStruct((_B, 1, _T1 * _T2), jnp.int32),
        jax.ShapeDtypeStruct((_B, _T1, 2 * _M), jnp.int32),
    ],
)

_bc_in_specs = [
    _batched((_T1, _M)),
    _batched((_T1, _M)),
    _batched((_T1, 3 * _M)),
    _full((3 * _M, _M)),
    _full((1, 3 * _M)),
    _full((_M, _M)),
    _full((1, _M)),
]

_stage_b = pl.pallas_call(
    functools.partial(_stage_bc_body, True),
    grid=(_B,),
    in_specs=_bc_in_specs,
    out_specs=[_batched((_T1, 2 * _M))],
    out_shape=[jax.ShapeDtypeStruct((_B, _T1, 2 * _M), jnp.int32)],
)

_stage_c = pl.pallas_call(
    functools.partial(_stage_bc_body, False),
    grid=(_B,),
    in_specs=_bc_in_specs,
    out_specs=[_batched((_T1, _M))],
    out_shape=[jax.ShapeDtypeStruct((_B, _T1, _M), jnp.float32)],
)


def _unpack2(wv):
    lo = lax.bitcast_convert_type(lax.shift_left(wv, 16), jnp.float32)
    hi = lax.bitcast_convert_type(jnp.bitwise_and(wv, _HI), jnp.float32)
    return lo, hi


def _pack2(a, b):
    ua = lax.bitcast_convert_type(a, jnp.int32)
    ta = ua + 0x7FFF + jnp.bitwise_and(lax.shift_right_logical(ua, 16), 1)
    ub = lax.bitcast_convert_type(b, jnp.int32)
    tb = ub + 0x7FFF + jnp.bitwise_and(lax.shift_right_logical(ub, 16), 1)
    return jnp.bitwise_or(lax.shift_right_logical(ta, 16),
                          jnp.bitwise_and(tb, _HI))


def _sc_gather_body(table_hbm, midx_hbm, xf_hbm, out_hbm,
                    idx_v, xf_v, rows_v, out_v, spm, sem):
    s_id = lax.axis_index("s")
    c_id = lax.axis_index("c")
    b = s_id // 4
    pltpu.sync_copy(table_hbm.at[c_id, pl.ds(s_id * 256, 256)],
                    spm.at[pl.ds(s_id * 256, 256)])

    @pl.when(s_id == 0)
    def _():
        pltpu.sync_copy(table_hbm.at[c_id, pl.ds(4096, _HALF - 4096)],
                        spm.at[pl.ds(4096, _HALF - 4096)])

    pltpu.sync_copy(midx_hbm.at[c_id, s_id], idx_v)
    pltpu.sync_copy(xf_hbm.at[b], xf_v)
    plsc.subcore_barrier()

    def compute_chunk(g, buf):
        def node_body(n8, _):
            def child_body(k, acc):
                row = n8 * _T2 + k
                new = list(acc)
                for blk in range(2):
                    lo, hi = _unpack2(buf[row, pl.ds(16 * blk, 16)])
                    new[2 * blk] = acc[2 * blk] + lo
                    new[2 * blk + 1] = acc[2 * blk + 1] + hi
                ys, cs = [], []
                for blk in range(2, 4):
                    ys.extend(_unpack2(buf[row, pl.ds(16 * blk, 16)]))
                for blk in range(4, 6):
                    cs.extend(_unpack2(buf[row, pl.ds(16 * blk, 16)]))
                for s in range(4):
                    xv = xf_v[k, pl.ds(16 * s, 16)]
                    f = 1.0 / (1.0 + jnp.exp(-(ys[s] + xv)))
                    new[4 + s] = acc[4 + s] + f * cs[s]
                return tuple(new)

            zero = jnp.zeros((16,), jnp.float32)
            acc = lax.fori_loop(0, _T2, child_body, (zero,) * 8)
            node = g * _CHUNK_NODES + n8
            for s in range(4):
                out_v[node, pl.ds(16 * s, 16)] = _pack2(acc[s], acc[4 + s])
            return 0

        lax.fori_loop(0, _CHUNK_NODES, node_body, 0)

    sub = _CHUNK_ROWS // _SUBS
    copies = [[None] * _SUBS, [None] * _SUBS]

    def fire(g, buf):
        for s in range(_SUBS):
            copies[buf][s] = pltpu.async_copy(
                spm.at[idx_v.at[g, pl.ds(s * sub, sub)]],
                rows_v.at[buf, pl.ds(s * sub, sub)], sem.at[buf])

    fire(0, 0)
    for g in range(_NCHUNK):
        cur = g % 2
        if g + 1 < _NCHUNK:
            fire(g + 1, (g + 1) % 2)
        for s in range(_SUBS):
            copies[cur][s].wait()
        compute_chunk(g, rows_v.at[cur])
    pltpu.sync_copy(out_v,
                    out_hbm.at[c_id, pl.ds(s_id * _NODES_PER_S, _NODES_PER_S)])


@functools.cache
def _get_sc_gather():
    return functools.partial(
        pl.kernel,
        mesh=plsc.VectorSubcoreMesh(core_axis_name="c", subcore_axis_name="s"),
        out_type=jax.ShapeDtypeStruct((2, _NPC, _M), jnp.int32),
        scratch_types=[
            pltpu.VMEM((_NCHUNK, _CHUNK_ROWS), jnp.int32),
            pltpu.VMEM((_T2, _M), jnp.float32),
            pltpu.VMEM((2, _CHUNK_ROWS, _TWW), jnp.int32),
            pltpu.VMEM((_NODES_PER_S, _M), jnp.int32),
            pltpu.VMEM_SHARED((_HALF, _TWW), jnp.int32),
            pltpu.SemaphoreType.DMA((2,)),
        ],
    )(_sc_gather_body)


def _build_table(packed_rows):
    pad = jnp.zeros((_B, 2, _TWW), jnp.int32)
    tab = jnp.concatenate([pad, packed_rows], axis=1).reshape(_ROWS, _TWW)
    ztail = jnp.zeros((_ROWS_PAD - _ROWS, _TWW), jnp.int32)
    return jnp.concatenate([tab, ztail], axis=0).reshape(2, _HALF, _TWW)


def _join_parts(parts):
    out_i32 = jnp.concatenate(parts, axis=1)
    return (out_i32[0].reshape(_B, _T1, _M),
            out_i32[1].reshape(_B, _T1, _M))


def kernel(token_encodings, trees, child_mask, max_depth,
           Wx, bx, Wh_iou, bh_iou, Wh_f, bh_f):
    del max_depth
    trees_f = trees.reshape(_B, 1, _T1 * _T2).astype(jnp.int32)
    cm_f = child_mask.reshape(_B, 1, _T1 * _T2)
    bx2 = bx.reshape(1, 4 * _M)
    bhiou2 = bh_iou.reshape(1, 3 * _M)
    bhf2 = bh_f.reshape(1, _M)

    x_iou, xf_sub, midxa, midxb, packed = _stage_a(
        token_encodings, trees_f, cm_f, Wx, bx2, bhiou2, Wh_f, bhf2)
    table = _build_table(packed)
    midx_h = [jnp.stack([m.reshape(2, _NS, _NCHUNK, _CHUNK_ROWS)[h]
                         for m in (midxa, midxb)])
              for h in range(2)]
    xf_h = [xf_sub[h * 4:(h + 1) * 4] for h in range(2)]

    sc_gather = _get_sc_gather()
    for level in range(1, _DEPTH):
        parts = [sc_gather(table, midx_h[h], xf_h[h]) for h in range(2)]
        p0, p1 = _join_parts(parts)
        if level < _DEPTH - 1:
            (packed,) = _stage_b(p0, p1, x_iou, Wh_iou, bhiou2, Wh_f, bhf2)
            table = _build_table(packed)
        else:
            (h,) = _stage_c(p0, p1, x_iou, Wh_iou, bhiou2, Wh_f, bhf2)
    return h


# speedup vs baseline: 1.2824x; 1.0104x over previous
import functools

import jax
import jax.numpy as jnp
from jax import lax
from jax.experimental import pallas as pl
from jax.experimental.pallas import tpu as pltpu
from jax.experimental.pallas import tpu_sc as plsc

_B = 8
_T1 = 1024
_T2 = 16
_IN = 128
_M = 64
_DEPTH = 4
_ROWS = _B * (_T1 + 2)
_ROWS_PAD = 8320
_HALF = _ROWS_PAD // 2
_ZB = 4100
_N = _B * _T1
_NS = 16
_NPC = _N // 2
_NODES_PER_S = _NPC // _NS
_CHUNK_NODES = 8
_CHUNK_ROWS = _CHUNK_NODES * _T2
_NCHUNK = _NODES_PER_S // _CHUNK_NODES
_SUBS = 2
_TWW = 128
_HI = -65536


def _sigmoid(x):
    return jax.nn.sigmoid(x)


def _renorm_scale(x):
    n = jnp.sqrt(jnp.sum(x * x, axis=-1, keepdims=True))
    return jnp.where(n > 2.0, 2.0 / (n + 1e-7), 1.0)


def _tc_pack_words(y):
    def rne(u):
        return u + 0x7FFF + jnp.bitwise_and(lax.shift_right_logical(u, 16), 1)
    words = []
    for k in range(8):
        ua = rne(lax.bitcast_convert_type(y[:, 32 * k:32 * k + 16],
                                          jnp.int32))
        ub = rne(lax.bitcast_convert_type(y[:, 32 * k + 16:32 * k + 32],
                                          jnp.int32))
        words.append(jnp.bitwise_or(lax.shift_right_logical(ua, 16),
                                    jnp.bitwise_and(ub, _HI)))
    return jnp.concatenate(words, axis=-1)


def _tc_unpack_words(wv):
    lo = lax.bitcast_convert_type(lax.shift_left(wv, 16), jnp.float32)
    hi = lax.bitcast_convert_type(jnp.bitwise_and(wv, _HI), jnp.float32)
    return lo, hi


def _stage_a_body(te_ref, trees_ref, cm_ref, wx_ref, bx_ref, bhiou_ref,
                  whf_ref, bhf_ref,
                  xiou_ref, xfsub_ref, midxa_ref, midxb_ref, tab_ref):
    m = _M
    te = te_ref[0]
    x = lax.dot_general(te, wx_ref[...], (((1,), (1,)), ((), ())),
                        preferred_element_type=jnp.float32) + bx_ref[0]
    xiou_ref[0] = x[:, :3 * m]
    xfsub_ref[0] = x[:_T2, 3 * m:]
    bh = bhiou_ref[0]
    i = _sigmoid(x[:, :m] + bh[:m])
    o = _sigmoid(x[:, m:2 * m] + bh[m:2 * m])
    u = jnp.tanh(x[:, 2 * m:3 * m] + bh[2 * m:3 * m])
    c = i * u
    h = o * jnp.tanh(c)
    th = h * _renorm_scale(h)
    tc = c * _renorm_scale(c)
    yh = lax.dot_general(th, whf_ref[...], (((1,), (1,)), ((), ())),
                         preferred_element_type=jnp.float32) + bhf_ref[0]
    zp = jnp.zeros((_T1, _M), jnp.float32)
    tab_ref[0] = _tc_pack_words(jnp.concatenate([th, yh, tc, zp], axis=-1))
    t = jnp.where(cm_ref[0] > 0.0, trees_ref[0], 0)
    midxa_ref[0] = jnp.where(t < _HALF, t, 0)
    midxb_ref[0] = jnp.where(t >= _HALF, t - _HALF, _ZB)


def _stage_bc_body(make_table, p0_ref, p1_ref, xiou_ref,
                   whiou_ref, bhiou_ref, whf_ref, bhf_ref, *out_refs):
    m = _M
    hs0, fc0 = _tc_unpack_words(p0_ref[0])
    hs1, fc1 = _tc_unpack_words(p1_ref[0])
    hs = hs0 + hs1
    fc = fc0 + fc1
    s = xiou_ref[0] + lax.dot_general(
        hs, whiou_ref[...], (((1,), (1,)), ((), ())),
        preferred_element_type=jnp.float32) + bhiou_ref[0]
    i = _sigmoid(s[:, :m])
    o = _sigmoid(s[:, m:2 * m])
    u = jnp.tanh(s[:, 2 * m:])
    c = i * u + fc
    h = o * jnp.tanh(c)
    if make_table:
        th = h * _renorm_scale(h)
        tc = c * _renorm_scale(c)
        yh = lax.dot_general(th, whf_ref[...], (((1,), (1,)), ((), ())),
                             preferred_element_type=jnp.float32) + bhf_ref[0]
        zp = jnp.zeros((_T1, _M), jnp.float32)
        out_refs[0][0] = _tc_pack_words(
            jnp.concatenate([th, yh, tc, zp], axis=-1))
    else:
        out_refs[0][0] = h


def _full(shape):
    return pl.BlockSpec(shape, lambda b: (0,) * len(shape))


def _batched(shape):
    return pl.BlockSpec((1,) + shape, lambda b: (b,) + (0,) * len(shape))


_stage_a = pl.pallas_call(
    _stage_a_body,
    grid=(_B,),
    in_specs=[
        _batched((_T1, _IN)),
        _batched((1, _T1 * _T2)),
        _batched((1, _T1 * _T2)),
        _full((4 * _M, _IN)),
        _full((1, 4 * _M)),
        _full((1, 3 * _M)),
        _full((_M, _M)),
        _full((1, _M)),
    ],
    out_specs=[
        _batched((_T1, 3 * _M)),
        _batched((_T2, _M)),
        _batched((1, _T1 * _T2)),
        _batched((1, _T1 * _T2)),
        _batched((_T1, 2 * _M)),
    ],
    out_shape=[
        jax.ShapeDtypeStruct((_B, _T1, 3 * _M), jnp.float32),
        jax.ShapeDtypeStruct((_B, _T2, _M), jnp.float32),
        jax.ShapeDtypeStruct((_B, 1, _T1 * _T2), jnp.int32),
        jax.ShapeDtypeStruct((_B, 1, _T1 * _T2), jnp.int32),
        jax.ShapeDtypeStruct((_B, _T1, 2 * _M), jnp.int32),
    ],
)

_bc_in_specs = [
    _batched((_T1, _M)),
    _batched((_T1, _M)),
    _batched((_T1, 3 * _M)),
    _full((3 * _M, _M)),
    _full((1, 3 * _M)),
    _full((_M, _M)),
    _full((1, _M)),
]

_stage_b = pl.pallas_call(
    functools.partial(_stage_bc_body, True),
    grid=(_B,),
    in_specs=_bc_in_specs,
    out_specs=[_batched((_T1, 2 * _M))],
    out_shape=[jax.ShapeDtypeStruct((_B, _T1, 2 * _M), jnp.int32)],
)

_stage_c = pl.pallas_call(
    functools.partial(_stage_bc_body, False),
    grid=(_B,),
    in_specs=_bc_in_specs,
    out_specs=[_batched((_T1, _M))],
    out_shape=[jax.ShapeDtypeStruct((_B, _T1, _M), jnp.float32)],
)


def _unpack2(wv):
    lo = lax.bitcast_convert_type(lax.shift_left(wv, 16), jnp.float32)
    hi = lax.bitcast_convert_type(jnp.bitwise_and(wv, _HI), jnp.float32)
    return lo, hi


def _pack2(a, b):
    ua = lax.bitcast_convert_type(a, jnp.int32)
    ta = ua + 0x7FFF + jnp.bitwise_and(lax.shift_right_logical(ua, 16), 1)
    ub = lax.bitcast_convert_type(b, jnp.int32)
    tb = ub + 0x7FFF + jnp.bitwise_and(lax.shift_right_logical(ub, 16), 1)
    return jnp.bitwise_or(lax.shift_right_logical(ta, 16),
                          jnp.bitwise_and(tb, _HI))


def _sc_gather_body(table_hbm, midx_hbm, xf_hbm, out_hbm,
                    idx_v, xf_v, rows_v, out_v, spm, sem):
    s_id = lax.axis_index("s")
    c_id = lax.axis_index("c")
    b = s_id // 4
    pltpu.sync_copy(table_hbm.at[c_id, pl.ds(s_id * 256, 256)],
                    spm.at[pl.ds(s_id * 256, 256)])

    @pl.when(s_id == 0)
    def _():
        pltpu.sync_copy(table_hbm.at[c_id, pl.ds(4096, _HALF - 4096)],
                        spm.at[pl.ds(4096, _HALF - 4096)])

    pltpu.sync_copy(midx_hbm.at[c_id, s_id], idx_v)
    pltpu.sync_copy(xf_hbm.at[b], xf_v)
    plsc.subcore_barrier()

    def compute_chunk(g, buf):
        def node_body(n8, _):
            def child_body(k, acc):
                row = n8 * _T2 + k
                new = list(acc)
                for blk in range(2):
                    lo, hi = _unpack2(buf[row, pl.ds(16 * blk, 16)])
                    new[2 * blk] = acc[2 * blk] + lo
                    new[2 * blk + 1] = acc[2 * blk + 1] + hi
                ys, cs = [], []
                for blk in range(2, 4):
                    ys.extend(_unpack2(buf[row, pl.ds(16 * blk, 16)]))
                for blk in range(4, 6):
                    cs.extend(_unpack2(buf[row, pl.ds(16 * blk, 16)]))
                for s in range(4):
                    xv = xf_v[k, pl.ds(16 * s, 16)]
                    f = 1.0 / (1.0 + jnp.exp(-(ys[s] + xv)))
                    new[4 + s] = acc[4 + s] + f * cs[s]
                return tuple(new)

            zero = jnp.zeros((16,), jnp.float32)
            acc = lax.fori_loop(0, _T2, child_body, (zero,) * 8)
            node = g * _CHUNK_NODES + n8
            for s in range(4):
                out_v[node, pl.ds(16 * s, 16)] = _pack2(acc[s], acc[4 + s])
            return 0

        lax.fori_loop(0, _CHUNK_NODES, node_body, 0)

    sub = _CHUNK_ROWS // _SUBS
    copies = [[None] * _SUBS, [None] * _SUBS]

    def fire(g, buf):
        for s in range(_SUBS):
            copies[buf][s] = pltpu.async_copy(
                spm.at[idx_v.at[g, pl.ds(s * sub, sub)]],
                rows_v.at[buf, pl.ds(s * sub, sub)], sem.at[buf])

    fire(0, 0)
    for g in range(_NCHUNK):
        cur = g % 2
        if g + 1 < _NCHUNK:
            fire(g + 1, (g + 1) % 2)
        for s in range(_SUBS):
            copies[cur][s].wait()
        compute_chunk(g, rows_v.at[cur])
    pltpu.sync_copy(out_v,
                    out_hbm.at[c_id, pl.ds(s_id * _NODES_PER_S, _NODES_PER_S)])


@functools.cache
def _get_sc_gather():
    return functools.partial(
        pl.kernel,
        mesh=plsc.VectorSubcoreMesh(core_axis_name="c", subcore_axis_name="s"),
        out_type=jax.ShapeDtypeStruct((2, _NPC, _M), jnp.int32),
        scratch_types=[
            pltpu.VMEM((_NCHUNK, _CHUNK_ROWS), jnp.int32),
            pltpu.VMEM((_T2, _M), jnp.float32),
            pltpu.VMEM((2, _CHUNK_ROWS, _TWW), jnp.int32),
            pltpu.VMEM((_NODES_PER_S, _M), jnp.int32),
            pltpu.VMEM_SHARED((_HALF, _TWW), jnp.int32),
            pltpu.SemaphoreType.DMA((2,)),
        ],
    )(_sc_gather_body)


def _build_table(packed_rows):
    pad = jnp.zeros((_B, 2, _TWW), jnp.int32)
    tab = jnp.concatenate([pad, packed_rows], axis=1).reshape(_ROWS, _TWW)
    ztail = jnp.zeros((_ROWS_PAD - _ROWS, _TWW), jnp.int32)
    return jnp.concatenate([tab, ztail], axis=0).reshape(2, _HALF, _TWW)


def _join_parts(parts):
    out_i32 = jnp.concatenate(parts, axis=1)
    return (out_i32[0].reshape(_B, _T1, _M),
            out_i32[1].reshape(_B, _T1, _M))


def kernel(token_encodings, trees, child_mask, max_depth,
           Wx, bx, Wh_iou, bh_iou, Wh_f, bh_f):
    del max_depth
    trees_f = trees.reshape(_B, 1, _T1 * _T2).astype(jnp.int32)
    cm_f = child_mask.reshape(_B, 1, _T1 * _T2)
    bx2 = bx.reshape(1, 4 * _M)
    bhiou2 = bh_iou.reshape(1, 3 * _M)
    bhf2 = bh_f.reshape(1, _M)

    x_iou, xf_sub, midxa, midxb, packed = _stage_a(
        token_encodings, trees_f, cm_f, Wx, bx2, bhiou2, Wh_f, bhf2)
    table = _build_table(packed)
    midx_h = [jnp.stack([m.reshape(2, _NS, _NCHUNK, _CHUNK_ROWS)[h]
                         for m in (midxa, midxb)])
              for h in range(2)]
    xf_h = [xf_sub[h * 4:(h + 1) * 4] for h in range(2)]

    sc_gather = _get_sc_gather()
    for level in range(1, _DEPTH):
        parts = [sc_gather(table, midx_h[h], xf_h[h]) for h in range(2)]
        p0, p1 = _join_parts(parts)
        if level < _DEPTH - 1:
            (packed,) = _stage_b(p0, p1, x_iou, Wh_iou, bhiou2, Wh_f, bhf2)
            table = _build_table(packed)
        else:
            (h,) = _stage_c(p0, p1, x_iou, Wh_iou, bhiou2, Wh_f, bhf2)
    return h


# speedup vs baseline: 1.2843x; 1.0015x over previous
import functools

import jax
import jax.numpy as jnp
from jax import lax
from jax.experimental import pallas as pl
from jax.experimental.pallas import tpu as pltpu
from jax.experimental.pallas import tpu_sc as plsc

_B = 8
_T1 = 1024
_T2 = 16
_IN = 128
_M = 64
_DEPTH = 4
_ROWS = _B * (_T1 + 2)
_ROWS_PAD = 8320
_HALF = _ROWS_PAD // 2
_ZB = 4100
_N = _B * _T1
_NS = 16
_NPC = _N // 2
_NODES_PER_S = _NPC // _NS
_CHUNK_NODES = 8
_CHUNK_ROWS = _CHUNK_NODES * _T2
_NCHUNK = _NODES_PER_S // _CHUNK_NODES
_SUBS = 1
_TWW = 128
_HI = -65536


def _sigmoid(x):
    return jax.nn.sigmoid(x)


def _renorm_scale(x):
    n = jnp.sqrt(jnp.sum(x * x, axis=-1, keepdims=True))
    return jnp.where(n > 2.0, 2.0 / (n + 1e-7), 1.0)


def _tc_pack_words(y):
    def rne(u):
        return u + 0x7FFF + jnp.bitwise_and(lax.shift_right_logical(u, 16), 1)
    words = []
    for k in range(8):
        ua = rne(lax.bitcast_convert_type(y[:, 32 * k:32 * k + 16],
                                          jnp.int32))
        ub = rne(lax.bitcast_convert_type(y[:, 32 * k + 16:32 * k + 32],
                                          jnp.int32))
        words.append(jnp.bitwise_or(lax.shift_right_logical(ua, 16),
                                    jnp.bitwise_and(ub, _HI)))
    return jnp.concatenate(words, axis=-1)


def _tc_unpack_words(wv):
    lo = lax.bitcast_convert_type(lax.shift_left(wv, 16), jnp.float32)
    hi = lax.bitcast_convert_type(jnp.bitwise_and(wv, _HI), jnp.float32)
    return lo, hi


def _stage_a_body(te_ref, trees_ref, cm_ref, wx_ref, bx_ref, bhiou_ref,
                  whf_ref, bhf_ref,
                  xiou_ref, xfsub_ref, midxa_ref, midxb_ref, tab_ref):
    m = _M
    te = te_ref[0]
    x = lax.dot_general(te, wx_ref[...], (((1,), (1,)), ((), ())),
                        preferred_element_type=jnp.float32) + bx_ref[0]
    xiou_ref[0] = x[:, :3 * m]
    xfsub_ref[0] = x[:_T2, 3 * m:]
    bh = bhiou_ref[0]
    i = _sigmoid(x[:, :m] + bh[:m])
    o = _sigmoid(x[:, m:2 * m] + bh[m:2 * m])
    u = jnp.tanh(x[:, 2 * m:3 * m] + bh[2 * m:3 * m])
    c = i * u
    h = o * jnp.tanh(c)
    th = h * _renorm_scale(h)
    tc = c * _renorm_scale(c)
    yh = lax.dot_general(th, whf_ref[...], (((1,), (1,)), ((), ())),
                         preferred_element_type=jnp.float32) + bhf_ref[0]
    zp = jnp.zeros((_T1, _M), jnp.float32)
    tab_ref[0] = _tc_pack_words(jnp.concatenate([th, yh, tc, zp], axis=-1))
    t = jnp.where(cm_ref[0] > 0.0, trees_ref[0], 0)
    midxa_ref[0] = jnp.where(t < _HALF, t, 0)
    midxb_ref[0] = jnp.where(t >= _HALF, t - _HALF, _ZB)


def _stage_bc_body(make_table, p0_ref, p1_ref, xiou_ref,
                   whiou_ref, bhiou_ref, whf_ref, bhf_ref, *out_refs):
    m = _M
    hs0, fc0 = _tc_unpack_words(p0_ref[0])
    hs1, fc1 = _tc_unpack_words(p1_ref[0])
    hs = hs0 + hs1
    fc = fc0 + fc1
    s = xiou_ref[0] + lax.dot_general(
        hs, whiou_ref[...], (((1,), (1,)), ((), ())),
        preferred_element_type=jnp.float32) + bhiou_ref[0]
    i = _sigmoid(s[:, :m])
    o = _sigmoid(s[:, m:2 * m])
    u = jnp.tanh(s[:, 2 * m:])
    c = i * u + fc
    h = o * jnp.tanh(c)
    if make_table:
        th = h * _renorm_scale(h)
        tc = c * _renorm_scale(c)
        yh = lax.dot_general(th, whf_ref[...], (((1,), (1,)), ((), ())),
                             preferred_element_type=jnp.float32) + bhf_ref[0]
        zp = jnp.zeros((_T1, _M), jnp.float32)
        out_refs[0][0] = _tc_pack_words(
            jnp.concatenate([th, yh, tc, zp], axis=-1))
    else:
        out_refs[0][0] = h


def _full(shape):
    return pl.BlockSpec(shape, lambda b: (0,) * len(shape))


def _batched(shape):
    return pl.BlockSpec((1,) + shape, lambda b: (b,) + (0,) * len(shape))


_stage_a = pl.pallas_call(
    _stage_a_body,
    grid=(_B,),
    in_specs=[
        _batched((_T1, _IN)),
        _batched((1, _T1 * _T2)),
        _batched((1, _T1 * _T2)),
        _full((4 * _M, _IN)),
        _full((1, 4 * _M)),
        _full((1, 3 * _M)),
        _full((_M, _M)),
        _full((1, _M)),
    ],
    out_specs=[
        _batched((_T1, 3 * _M)),
        _batched((_T2, _M)),
        _batched((1, _T1 * _T2)),
        _batched((1, _T1 * _T2)),
        _batched((_T1, 2 * _M)),
    ],
    out_shape=[
        jax.ShapeDtypeStruct((_B, _T1, 3 * _M), jnp.float32),
        jax.ShapeDtypeStruct((_B, _T2, _M), jnp.float32),
        jax.ShapeDtypeStruct((_B, 1, _T1 * _T2), jnp.int32),
        jax.ShapeDtypeStruct((_B, 1, _T1 * _T2), jnp.int32),
        jax.ShapeDtypeStruct((_B, _T1, 2 * _M), jnp.int32),
    ],
)

_bc_in_specs = [
    _batched((_T1, _M)),
    _batched((_T1, _M)),
    _batched((_T1, 3 * _M)),
    _full((3 * _M, _M)),
    _full((1, 3 * _M)),
    _full((_M, _M)),
    _full((1, _M)),
]

_stage_b = pl.pallas_call(
    functools.partial(_stage_bc_body, True),
    grid=(_B,),
    in_specs=_bc_in_specs,
    out_specs=[_batched((_T1, 2 * _M))],
    out_shape=[jax.ShapeDtypeStruct((_B, _T1, 2 * _M), jnp.int32)],
)

_stage_c = pl.pallas_call(
    functools.partial(_stage_bc_body, False),
    grid=(_B,),
    in_specs=_bc_in_specs,
    out_specs=[_batched((_T1, _M))],
    out_shape=[jax.ShapeDtypeStruct((_B, _T1, _M), jnp.float32)],
)


def _unpack2(wv):
    lo = lax.bitcast_convert_type(lax.shift_left(wv, 16), jnp.float32)
    hi = lax.bitcast_convert_type(jnp.bitwise_and(wv, _HI), jnp.float32)
    return lo, hi


def _pack2(a, b):
    ua = lax.bitcast_convert_type(a, jnp.int32)
    ta = ua + 0x7FFF + jnp.bitwise_and(lax.shift_right_logical(ua, 16), 1)
    ub = lax.bitcast_convert_type(b, jnp.int32)
    tb = ub + 0x7FFF + jnp.bitwise_and(lax.shift_right_logical(ub, 16), 1)
    return jnp.bitwise_or(lax.shift_right_logical(ta, 16),
                          jnp.bitwise_and(tb, _HI))


def _sc_gather_body(table_hbm, midx_hbm, xf_hbm, out_hbm,
                    idx_v, xf_v, rows_v, out_v, spm, sem):
    s_id = lax.axis_index("s")
    c_id = lax.axis_index("c")
    b = s_id // 4
    pltpu.sync_copy(table_hbm.at[c_id, pl.ds(s_id * 256, 256)],
                    spm.at[pl.ds(s_id * 256, 256)])

    @pl.when(s_id == 0)
    def _():
        pltpu.sync_copy(table_hbm.at[c_id, pl.ds(4096, _HALF - 4096)],
                        spm.at[pl.ds(4096, _HALF - 4096)])

    pltpu.sync_copy(midx_hbm.at[c_id, s_id], idx_v)
    pltpu.sync_copy(xf_hbm.at[b], xf_v)
    plsc.subcore_barrier()

    def compute_chunk(g, buf):
        def node_body(n8, _):
            def child_body(k, acc):
                row = n8 * _T2 + k
                new = list(acc)
                for blk in range(2):
                    lo, hi = _unpack2(buf[row, pl.ds(16 * blk, 16)])
                    new[2 * blk] = acc[2 * blk] + lo
                    new[2 * blk + 1] = acc[2 * blk + 1] + hi
                ys, cs = [], []
                for blk in range(2, 4):
                    ys.extend(_unpack2(buf[row, pl.ds(16 * blk, 16)]))
                for blk in range(4, 6):
                    cs.extend(_unpack2(buf[row, pl.ds(16 * blk, 16)]))
                for s in range(4):
                    xv = xf_v[k, pl.ds(16 * s, 16)]
                    f = 1.0 / (1.0 + jnp.exp(-(ys[s] + xv)))
                    new[4 + s] = acc[4 + s] + f * cs[s]
                return tuple(new)

            zero = jnp.zeros((16,), jnp.float32)
            acc = lax.fori_loop(0, _T2, child_body, (zero,) * 8)
            node = g * _CHUNK_NODES + n8
            for s in range(4):
                out_v[node, pl.ds(16 * s, 16)] = _pack2(acc[s], acc[4 + s])
            return 0

        lax.fori_loop(0, _CHUNK_NODES, node_body, 0)

    sub = _CHUNK_ROWS // _SUBS
    copies = [[None] * _SUBS, [None] * _SUBS]

    def fire(g, buf):
        for s in range(_SUBS):
            copies[buf][s] = pltpu.async_copy(
                spm.at[idx_v.at[g, pl.ds(s * sub, sub)]],
                rows_v.at[buf, pl.ds(s * sub, sub)], sem.at[buf])

    fire(0, 0)
    for g in range(_NCHUNK):
        cur = g % 2
        if g + 1 < _NCHUNK:
            fire(g + 1, (g + 1) % 2)
        for s in range(_SUBS):
            copies[cur][s].wait()
        compute_chunk(g, rows_v.at[cur])
    pltpu.sync_copy(out_v,
                    out_hbm.at[c_id, pl.ds(s_id * _NODES_PER_S, _NODES_PER_S)])


@functools.cache
def _get_sc_gather():
    return functools.partial(
        pl.kernel,
        mesh=plsc.VectorSubcoreMesh(core_axis_name="c", subcore_axis_name="s"),
        out_type=jax.ShapeDtypeStruct((2, _NPC, _M), jnp.int32),
        scratch_types=[
            pltpu.VMEM((_NCHUNK, _CHUNK_ROWS), jnp.int32),
            pltpu.VMEM((_T2, _M), jnp.float32),
            pltpu.VMEM((2, _CHUNK_ROWS, _TWW), jnp.int32),
            pltpu.VMEM((_NODES_PER_S, _M), jnp.int32),
            pltpu.VMEM_SHARED((_HALF, _TWW), jnp.int32),
            pltpu.SemaphoreType.DMA((2,)),
        ],
    )(_sc_gather_body)


def _build_table(packed_rows):
    pad = jnp.zeros((_B, 2, _TWW), jnp.int32)
    tab = jnp.concatenate([pad, packed_rows], axis=1).reshape(_ROWS, _TWW)
    ztail = jnp.zeros((_ROWS_PAD - _ROWS, _TWW), jnp.int32)
    return jnp.concatenate([tab, ztail], axis=0).reshape(2, _HALF, _TWW)


def _join_parts(parts):
    out_i32 = jnp.concatenate(parts, axis=1)
    return (out_i32[0].reshape(_B, _T1, _M),
            out_i32[1].reshape(_B, _T1, _M))


def kernel(token_encodings, trees, child_mask, max_depth,
           Wx, bx, Wh_iou, bh_iou, Wh_f, bh_f):
    del max_depth
    trees_f = trees.reshape(_B, 1, _T1 * _T2).astype(jnp.int32)
    cm_f = child_mask.reshape(_B, 1, _T1 * _T2)
    bx2 = bx.reshape(1, 4 * _M)
    bhiou2 = bh_iou.reshape(1, 3 * _M)
    bhf2 = bh_f.reshape(1, _M)

    x_iou, xf_sub, midxa, midxb, packed = _stage_a(
        token_encodings, trees_f, cm_f, Wx, bx2, bhiou2, Wh_f, bhf2)
    table = _build_table(packed)
    midx_h = [jnp.stack([m.reshape(2, _NS, _NCHUNK, _CHUNK_ROWS)[h]
                         for m in (midxa, midxb)])
              for h in range(2)]
    xf_h = [xf_sub[h * 4:(h + 1) * 4] for h in range(2)]

    sc_gather = _get_sc_gather()
    for level in range(1, _DEPTH):
        parts = [sc_gather(table, midx_h[h], xf_h[h]) for h in range(2)]
        p0, p1 = _join_parts(parts)
        if level < _DEPTH - 1:
            (packed,) = _stage_b(p0, p1, x_iou, Wh_iou, bhiou2, Wh_f, bhf2)
            table = _build_table(packed)
        else:
            (h,) = _stage_c(p0, p1, x_iou, Wh_iou, bhiou2, Wh_f, bhf2)
    return h
